# baseline (speedup 1.0000x reference)
import dataclasses
import functools

import jax
import jax.numpy as jnp
import numpy as np
from jax import lax
from jax.experimental import pallas as pl
from jax.experimental.pallas import tpu as pltpu
from jax.experimental.pallas import tpu_sc as plsc

N_NODES = 10000
N_EDGES = 160000
F = 128
K_RBF = 5
CUTOFF = 10.0
LOG2 = float(np.log(2.0))

NP = 10016
EP = 163840
CHUNK = 128
N_CHUNKS = EP // CHUNK
CHUNKS_PER_W = N_CHUNKS // 32
TILE = 1024
N_TILES = EP // TILE

RBF_WIDTH = float((0.5 / ((1.0 - np.exp(-CUTOFF)) / K_RBF)) ** 2)
RBF_STEP = float((np.exp(-CUTOFF) - 1.0) / (K_RBF - 1))


def _ssp(x):
    return jnp.maximum(x, 0.0) + jnp.log1p(jnp.exp(-jnp.abs(x))) - LOG2


def _node_embed_body(z_ref, emb_ref, wi_ref, bi_ref, wj_ref, bj_ref,
                     a_ref, b_ref):
    z = z_ref[...]
    ids = lax.broadcasted_iota(jnp.int32, (1, 24), 1)
    oh = (z == ids).astype(jnp.float32)
    x = jnp.dot(oh, emb_ref[...], preferred_element_type=jnp.float32)
    h = _ssp(x)
    a = jax.lax.dot_general(h, wi_ref[...], (((1,), (0,)), ((), ())),
                            precision=lax.Precision.HIGHEST) + bi_ref[...]
    b = jax.lax.dot_general(h, wj_ref[...], (((1,), (0,)), ((), ())),
                            precision=lax.Precision.HIGHEST) + bj_ref[...]
    a_ref[0:N_NODES, :] = a
    a_ref[N_NODES:NP, :] = jnp.zeros((NP - N_NODES, F), jnp.float32)
    b_ref[0:N_NODES, :] = b
    b_ref[N_NODES:NP, :] = jnp.zeros((NP - N_NODES, F), jnp.float32)


def _node_parts_body(p_ref, wi_ref, bi_ref, wj_ref, bj_ref, a_ref, b_ref):
    x = p_ref[0, 0:N_NODES, :] + p_ref[1, 0:N_NODES, :]
    h = _ssp(x)
    a = jax.lax.dot_general(h, wi_ref[...], (((1,), (0,)), ((), ())),
                            precision=lax.Precision.HIGHEST) + bi_ref[...]
    b = jax.lax.dot_general(h, wj_ref[...], (((1,), (0,)), ((), ())),
                            precision=lax.Precision.HIGHEST) + bj_ref[...]
    a_ref[0:N_NODES, :] = a
    a_ref[N_NODES:NP, :] = jnp.zeros((NP - N_NODES, F), jnp.float32)
    b_ref[0:N_NODES, :] = b
    b_ref[N_NODES:NP, :] = jnp.zeros((NP - N_NODES, F), jnp.float32)


def _final_body(p_ref, o_ref):
    o_ref[...] = p_ref[0, 0:N_NODES, :] + p_ref[1, 0:N_NODES, :]


_NODE_OUT = (jax.ShapeDtypeStruct((NP, F), jnp.float32),
             jax.ShapeDtypeStruct((NP, F), jnp.float32))


def _node_embed(z2d, emb_pad, wi, bi, wj, bj):
    return pl.pallas_call(_node_embed_body, out_shape=_NODE_OUT)(
        z2d, emb_pad, wi, bi, wj, bj)


def _node_parts(parts, wi, bi, wj, bj):
    return pl.pallas_call(_node_parts_body, out_shape=_NODE_OUT)(
        parts, wi, bi, wj, bj)


def _final_sum(parts):
    return pl.pallas_call(
        _final_body,
        out_shape=jax.ShapeDtypeStruct((N_NODES, F), jnp.float32))(parts)


_VMESH = plsc.VectorSubcoreMesh(core_axis_name="c", subcore_axis_name="s")

_SC_CP = pltpu.CompilerParams()
if "needs_layout_passes" in pltpu.CompilerParams.__dataclass_fields__:
    _SC_CP = dataclasses.replace(_SC_CP, needs_layout_passes=False)


@functools.partial(
    pl.kernel,
    out_type=(jax.ShapeDtypeStruct((EP, F), jnp.float32),
              jax.ShapeDtypeStruct((EP, F), jnp.float32),
              jax.ShapeDtypeStruct((EP,), jnp.float32)),
    mesh=_VMESH,
    scratch_types=[pltpu.VMEM((CHUNK,), jnp.int32),
                   pltpu.VMEM((CHUNK,), jnp.int32),
                   pltpu.VMEM((CHUNK, F), jnp.float32),
                   pltpu.VMEM((CHUNK,), jnp.float32),
                   pltpu.VMEM((NP,), jnp.float32),
                   pltpu.VMEM((NP,), jnp.float32),
                   pltpu.VMEM((NP,), jnp.float32),
                   pltpu.SemaphoreType.DMA],
    compiler_params=_SC_CP,
)
def _sc_gather(a_hbm, b_hbm, src_hbm, dst_hbm, px_hbm, py_hbm, pz_hbm,
               ari_hbm, bjg_hbm, d2_hbm,
               idxs_v, idxd_v, rows_v, d2_v, px_v, py_v, pz_v, sem):
    c = lax.axis_index("c")
    s = lax.axis_index("s")
    pltpu.sync_copy(px_hbm, px_v)
    pltpu.sync_copy(py_hbm, py_v)
    pltpu.sync_copy(pz_hbm, pz_v)
    base0 = (c * 640 + s * CHUNKS_PER_W) * CHUNK

    @pl.loop(0, CHUNKS_PER_W)
    def _(j):
        base = base0 + j * CHUNK
        pltpu.sync_copy(dst_hbm.at[pl.ds(base, CHUNK)], idxd_v)
        pltpu.async_copy(a_hbm.at[idxd_v], rows_v, sem).wait()
        pltpu.sync_copy(rows_v, ari_hbm.at[pl.ds(base, CHUNK)])
        pltpu.sync_copy(src_hbm.at[pl.ds(base, CHUNK)], idxs_v)
        pltpu.async_copy(b_hbm.at[idxs_v], rows_v, sem).wait()
        pltpu.sync_copy(rows_v, bjg_hbm.at[pl.ds(base, CHUNK)])
        for i in range(CHUNK // 16):
            s16 = idxs_v[pl.ds(i * 16, 16)]
            d16 = idxd_v[pl.ds(i * 16, 16)]
            dx = plsc.load_gather(px_v, [s16]) - plsc.load_gather(px_v, [d16])
            dy = plsc.load_gather(py_v, [s16]) - plsc.load_gather(py_v, [d16])
            dz = plsc.load_gather(pz_v, [s16]) - plsc.load_gather(pz_v, [d16])
            d2_v[pl.ds(i * 16, 16)] = dx * dx + dy * dy + dz * dz
        pltpu.sync_copy(d2_v, d2_hbm.at[pl.ds(base, CHUNK)])


NACC = 10240


@functools.partial(
    pl.kernel,
    out_type=jax.ShapeDtypeStruct((2, NACC, F), jnp.float32),
    mesh=_VMESH,
    scratch_types=[pltpu.VMEM_SHARED((NACC, F), jnp.float32),
                   pltpu.VMEM((CHUNK, F), jnp.float32),
                   pltpu.VMEM((CHUNK,), jnp.int32),
                   pltpu.SemaphoreType.DMA],
    compiler_params=_SC_CP,
)
def _sc_scatter(new_hbm, dst_hbm, zero_hbm, parts_hbm,
                acc_sp, rows_v, idx_v, sem):
    c = lax.axis_index("c")
    s = lax.axis_index("s")
    pltpu.sync_copy(zero_hbm, rows_v)
    for z in range(5):
        pltpu.sync_copy(rows_v, acc_sp.at[pl.ds(s * 640 + z * CHUNK, CHUNK)])
    plsc.subcore_barrier()

    @pl.loop(0, CHUNKS_PER_W)
    def _(j):
        base = (c * 640 + s * CHUNKS_PER_W + j) * CHUNK
        pltpu.sync_copy(dst_hbm.at[pl.ds(base, CHUNK)], idx_v)
        pltpu.sync_copy(new_hbm.at[pl.ds(base, CHUNK)], rows_v)
        pltpu.sync_copy(rows_v, acc_sp.at[idx_v], add=True)

    plsc.subcore_barrier()
    pltpu.sync_copy(acc_sp.at[pl.ds(s * 640, 640)],
                    parts_hbm.at[c, pl.ds(s * 640, 640)])


def _edge_body(ari_ref, bjg_ref, d2_ref, wk2f_ref, u_ref, wd_ref, bd_ref,
               w1a_ref, b1a_ref, w2a_ref, b2a_ref,
               w1b_ref, b1b_ref, w2b_ref, b2b_ref, out_ref):
    d2b = d2_ref[...]
    er = lax.broadcasted_iota(jnp.int32, (TILE, 8), 0)
    sc = lax.broadcasted_iota(jnp.int32, (TILE, 8), 1)
    sel = (er // F == sc).astype(jnp.float32)
    t = jax.lax.dot_general(sel, d2b, (((1,), (0,)), ((), ())),
                            precision=lax.Precision.HIGHEST)
    el = lax.broadcasted_iota(jnp.int32, (TILE, F), 0)
    lc = lax.broadcasted_iota(jnp.int32, (TILE, F), 1)
    lmask = (el % F == lc).astype(jnp.float32)
    d2col = jnp.sum(t * lmask, axis=1, keepdims=True)
    dist = jnp.sqrt(d2col + 1e-12)
    xq = dist * (1.0 / CUTOFF)
    x3 = xq * xq * xq
    x4 = x3 * xq
    x5 = x4 * xq
    cut = jnp.where(xq < 1.0, 1.0 - 6.0 * x5 + 15.0 * x4 - 10.0 * x3, 0.0)
    e_d = jnp.exp(-dist)
    kki = lax.broadcasted_iota(jnp.int32, (1, 8), 1)
    kk = kki.astype(jnp.float32)
    centers = jnp.where(kki < K_RBF, 1.0 + kk * RBF_STEP, 0.0)
    g8 = cut * jnp.exp(-RBF_WIDTH * (e_d - centers) ** 2)
    mm = functools.partial(jax.lax.dot_general,
                           dimension_numbers=(((1,), (0,)), ((), ())),
                           precision=lax.Precision.HIGHEST)
    g = mm(g8, wk2f_ref[...])
    ari = ari_ref[...]
    arj = g * bjg_ref[...]
    m = ari + arj
    res = ((w1a_ref, b1a_ref, w2a_ref, b2a_ref),
           (w1b_ref, b1b_ref, w2b_ref, b2b_ref))
    for w1, b1, w2, b2 in res:
        t = mm(_ssp(m), w1[...]) + b1[...]
        m = m + mm(t, w2[...]) + b2[...]
    m = _ssp(m)
    new = u_ref[...] * ari + mm(m, wd_ref[...]) + bd_ref[...]
    for w1, b1, w2, b2 in res:
        t = mm(_ssp(new), w1[...]) + b1[...]
        new = new + mm(t, w2[...]) + b2[...]
    out_ref[...] = new


def _edge_chain(ari, bjg, d2, wk2f8, u, wd, bd, w1a, b1a, w2a, b2a,
                w1b, b1b, w2b, b2b):
    full = lambda shape: pl.BlockSpec(shape, lambda i: (0,) * len(shape))
    return pl.pallas_call(
        _edge_body,
        grid=(N_TILES,),
        in_specs=[
            pl.BlockSpec((TILE, F), lambda i: (i, 0)),
            pl.BlockSpec((TILE, F), lambda i: (i, 0)),
            pl.BlockSpec((8, F), lambda i: (i, 0)),
            full((8, F)), full((1, F)), full((F, F)), full((1, F)),
            full((F, F)), full((1, F)), full((F, F)), full((1, F)),
            full((F, F)), full((1, F)), full((F, F)), full((1, F)),
        ],
        out_specs=pl.BlockSpec((TILE, F), lambda i: (i, 0)),
        out_shape=jax.ShapeDtypeStruct((EP, F), jnp.float32),
    )(ari, bjg, d2.reshape(EP // F, F), wk2f8, u, wd, bd,
      w1a, b1a, w2a, b2a, w1b, b1b, w2b, b2b)


def kernel(Z, pos, edge_index, emb_table, params):
    src = edge_index[0].astype(jnp.int32)
    dst = edge_index[1].astype(jnp.int32)
    pad = jnp.full((EP - N_EDGES,), N_NODES, jnp.int32)
    srcp = jnp.concatenate([src, pad])
    dstp = jnp.concatenate([dst, pad])
    posp = jnp.pad(pos.astype(jnp.float32), ((0, NP - N_NODES), (0, 0)))
    px, py, pz = posp[:, 0], posp[:, 1], posp[:, 2]
    z2d = Z.astype(jnp.int32).reshape(N_NODES, 1)
    emb_pad = jnp.pad(emb_table, ((0, 24 - emb_table.shape[0]), (0, 0)))
    zero_rows = jnp.zeros((CHUNK, F), jnp.float32)

    def block_edges(a, b, p):
        wk2f8 = jnp.pad(p["Wk2f"], ((0, 8 - K_RBF), (0, 0)))
        ari, bjg, d2 = _sc_gather(a, b, srcp, dstp, px, py, pz)
        new = _edge_chain(
            ari, bjg, d2, wk2f8, p["u"].reshape(1, F), p["Wd"],
            p["bd"].reshape(1, F),
            p["res"][0]["W1"], p["res"][0]["b1"].reshape(1, F),
            p["res"][0]["W2"], p["res"][0]["b2"].reshape(1, F),
            p["res"][1]["W1"], p["res"][1]["b1"].reshape(1, F),
            p["res"][1]["W2"], p["res"][1]["b2"].reshape(1, F))
        return _sc_scatter(new, dstp, zero_rows)

    p0, p1 = params[0], params[1]
    a, b = _node_embed(z2d, emb_pad, p0["Wi"], p0["bi"].reshape(1, F),
                       p0["Wj"], p0["bj"].reshape(1, F))
    parts = block_edges(a, b, p0)
    a, b = _node_parts(parts, p1["Wi"], p1["bi"].reshape(1, F),
                       p1["Wj"], p1["bj"].reshape(1, F))
    parts = block_edges(a, b, p1)
    return _final_sum(parts)


# speedup vs baseline: 1.7894x; 1.7894x over previous
import dataclasses
import functools

import jax
import jax.numpy as jnp
import numpy as np
from jax import lax
from jax.experimental import pallas as pl
from jax.experimental.pallas import tpu as pltpu
from jax.experimental.pallas import tpu_sc as plsc

N_NODES = 10000
N_EDGES = 160000
F = 128
K_RBF = 5
CUTOFF = 10.0
LOG2 = float(np.log(2.0))

NP = 10016
EP = 163840
CHUNK = 128
N_CHUNKS = EP // CHUNK
CHUNKS_PER_W = N_CHUNKS // 32
TILE = 1024
N_TILES = EP // TILE

RBF_WIDTH = float((0.5 / ((1.0 - np.exp(-CUTOFF)) / K_RBF)) ** 2)
RBF_STEP = float((np.exp(-CUTOFF) - 1.0) / (K_RBF - 1))


def _ssp(x):
    return jnp.maximum(x, 0.0) + jnp.log1p(jnp.exp(-jnp.abs(x))) - LOG2


def _node_embed_body(z_ref, emb_ref, wi_ref, bi_ref, wj_ref, bj_ref,
                     a_ref, b_ref):
    z = z_ref[...]
    ids = lax.broadcasted_iota(jnp.int32, (1, 24), 1)
    oh = (z == ids).astype(jnp.float32)
    x = jnp.dot(oh, emb_ref[...], preferred_element_type=jnp.float32)
    h = _ssp(x)
    a = jax.lax.dot_general(h, wi_ref[...], (((1,), (0,)), ((), ())),
                            precision=lax.Precision.HIGHEST) + bi_ref[...]
    b = jax.lax.dot_general(h, wj_ref[...], (((1,), (0,)), ((), ())),
                            precision=lax.Precision.HIGHEST) + bj_ref[...]
    a_ref[0:N_NODES, :] = a
    a_ref[N_NODES:NP, :] = jnp.zeros((NP - N_NODES, F), jnp.float32)
    b_ref[0:N_NODES, :] = b
    b_ref[N_NODES:NP, :] = jnp.zeros((NP - N_NODES, F), jnp.float32)


def _node_parts_body(p_ref, wi_ref, bi_ref, wj_ref, bj_ref, a_ref, b_ref):
    x = p_ref[0, 0:N_NODES, :] + p_ref[1, 0:N_NODES, :]
    h = _ssp(x)
    a = jax.lax.dot_general(h, wi_ref[...], (((1,), (0,)), ((), ())),
                            precision=lax.Precision.HIGHEST) + bi_ref[...]
    b = jax.lax.dot_general(h, wj_ref[...], (((1,), (0,)), ((), ())),
                            precision=lax.Precision.HIGHEST) + bj_ref[...]
    a_ref[0:N_NODES, :] = a
    a_ref[N_NODES:NP, :] = jnp.zeros((NP - N_NODES, F), jnp.float32)
    b_ref[0:N_NODES, :] = b
    b_ref[N_NODES:NP, :] = jnp.zeros((NP - N_NODES, F), jnp.float32)


def _final_body(p_ref, o_ref):
    o_ref[...] = p_ref[0, 0:N_NODES, :] + p_ref[1, 0:N_NODES, :]


_NODE_OUT = (jax.ShapeDtypeStruct((NP, F), jnp.float32),
             jax.ShapeDtypeStruct((NP, F), jnp.float32))


def _node_embed(z2d, emb_pad, wi, bi, wj, bj):
    return pl.pallas_call(_node_embed_body, out_shape=_NODE_OUT)(
        z2d, emb_pad, wi, bi, wj, bj)


def _node_parts(parts, wi, bi, wj, bj):
    return pl.pallas_call(_node_parts_body, out_shape=_NODE_OUT)(
        parts, wi, bi, wj, bj)


def _final_sum(parts):
    return pl.pallas_call(
        _final_body,
        out_shape=jax.ShapeDtypeStruct((N_NODES, F), jnp.float32))(parts)


_VMESH = plsc.VectorSubcoreMesh(core_axis_name="c", subcore_axis_name="s")

_SC_CP = pltpu.CompilerParams()
if "needs_layout_passes" in pltpu.CompilerParams.__dataclass_fields__:
    _SC_CP = dataclasses.replace(_SC_CP, needs_layout_passes=False)


@functools.partial(
    pl.kernel,
    out_type=(jax.ShapeDtypeStruct((EP, F), jnp.float32),
              jax.ShapeDtypeStruct((EP, F), jnp.float32),
              jax.ShapeDtypeStruct((EP,), jnp.float32)),
    mesh=_VMESH,
    scratch_types=[pltpu.VMEM((CHUNK,), jnp.int32),
                   pltpu.VMEM((CHUNK,), jnp.int32),
                   pltpu.VMEM((CHUNK, F), jnp.float32),
                   pltpu.VMEM((CHUNK,), jnp.float32),
                   pltpu.VMEM((NP,), jnp.float32),
                   pltpu.VMEM((NP,), jnp.float32),
                   pltpu.VMEM((NP,), jnp.float32),
                   pltpu.SemaphoreType.DMA],
    compiler_params=_SC_CP,
)
def _sc_gather(a_hbm, b_hbm, src_hbm, dst_hbm, px_hbm, py_hbm, pz_hbm,
               ari_hbm, bjg_hbm, d2_hbm,
               idxs_v, idxd_v, rows_v, d2_v, px_v, py_v, pz_v, sem):
    c = lax.axis_index("c")
    s = lax.axis_index("s")
    pltpu.sync_copy(px_hbm, px_v)
    pltpu.sync_copy(py_hbm, py_v)
    pltpu.sync_copy(pz_hbm, pz_v)
    base0 = (c * 640 + s * CHUNKS_PER_W) * CHUNK

    @pl.loop(0, CHUNKS_PER_W)
    def _(j):
        base = base0 + j * CHUNK
        pltpu.sync_copy(dst_hbm.at[pl.ds(base, CHUNK)], idxd_v)
        pltpu.async_copy(a_hbm.at[idxd_v], rows_v, sem).wait()
        pltpu.sync_copy(rows_v, ari_hbm.at[pl.ds(base, CHUNK)])
        pltpu.sync_copy(src_hbm.at[pl.ds(base, CHUNK)], idxs_v)
        pltpu.async_copy(b_hbm.at[idxs_v], rows_v, sem).wait()
        pltpu.sync_copy(rows_v, bjg_hbm.at[pl.ds(base, CHUNK)])
        for i in range(CHUNK // 16):
            s16 = idxs_v[pl.ds(i * 16, 16)]
            d16 = idxd_v[pl.ds(i * 16, 16)]
            dx = plsc.load_gather(px_v, [s16]) - plsc.load_gather(px_v, [d16])
            dy = plsc.load_gather(py_v, [s16]) - plsc.load_gather(py_v, [d16])
            dz = plsc.load_gather(pz_v, [s16]) - plsc.load_gather(pz_v, [d16])
            d2_v[pl.ds(i * 16, 16)] = dx * dx + dy * dy + dz * dz
        pltpu.sync_copy(d2_v, d2_hbm.at[pl.ds(base, CHUNK)])


NACC = 10240


@functools.partial(
    pl.kernel,
    out_type=jax.ShapeDtypeStruct((2, NACC, F), jnp.float32),
    mesh=_VMESH,
    scratch_types=[pltpu.VMEM_SHARED((NACC, F), jnp.float32),
                   pltpu.VMEM((CHUNK, F), jnp.float32),
                   pltpu.VMEM((CHUNK,), jnp.int32),
                   pltpu.SemaphoreType.DMA],
    compiler_params=_SC_CP,
)
def _sc_scatter(new_hbm, dst_hbm, zero_hbm, parts_hbm,
                acc_sp, rows_v, idx_v, sem):
    c = lax.axis_index("c")
    s = lax.axis_index("s")
    pltpu.sync_copy(zero_hbm, rows_v)
    for z in range(5):
        pltpu.sync_copy(rows_v, acc_sp.at[pl.ds(s * 640 + z * CHUNK, CHUNK)])
    plsc.subcore_barrier()

    @pl.loop(0, CHUNKS_PER_W)
    def _(j):
        base = (c * 640 + s * CHUNKS_PER_W + j) * CHUNK
        pltpu.sync_copy(dst_hbm.at[pl.ds(base, CHUNK)], idx_v)
        pltpu.sync_copy(new_hbm.at[pl.ds(base, CHUNK)], rows_v)
        pltpu.sync_copy(rows_v, acc_sp.at[idx_v], add=True)

    plsc.subcore_barrier()
    pltpu.sync_copy(acc_sp.at[pl.ds(s * 640, 640)],
                    parts_hbm.at[c, pl.ds(s * 640, 640)])


def _edge_body(ari_ref, bjg_ref, d2_ref, wk2f_ref, u_ref, wd_ref, bd_ref,
               w1a_ref, b1a_ref, w2a_ref, b2a_ref,
               w1b_ref, b1b_ref, w2b_ref, b2b_ref, out_ref):
    d2b = d2_ref[...]
    er = lax.broadcasted_iota(jnp.int32, (TILE, 8), 0)
    sc = lax.broadcasted_iota(jnp.int32, (TILE, 8), 1)
    sel = (er // F == sc).astype(jnp.float32)
    t = jax.lax.dot_general(sel, d2b, (((1,), (0,)), ((), ())),
                            precision=lax.Precision.HIGHEST)
    el = lax.broadcasted_iota(jnp.int32, (TILE, F), 0)
    lc = lax.broadcasted_iota(jnp.int32, (TILE, F), 1)
    lmask = (el % F == lc).astype(jnp.float32)
    d2col = jnp.sum(t * lmask, axis=1, keepdims=True)
    dist = jnp.sqrt(d2col + 1e-12)
    xq = dist * (1.0 / CUTOFF)
    x3 = xq * xq * xq
    x4 = x3 * xq
    x5 = x4 * xq
    cut = jnp.where(xq < 1.0, 1.0 - 6.0 * x5 + 15.0 * x4 - 10.0 * x3, 0.0)
    e_d = jnp.exp(-dist)
    kki = lax.broadcasted_iota(jnp.int32, (1, 8), 1)
    kk = kki.astype(jnp.float32)
    centers = jnp.where(kki < K_RBF, 1.0 + kk * RBF_STEP, 0.0)
    g8 = cut * jnp.exp(-RBF_WIDTH * (e_d - centers) ** 2)
    mm = functools.partial(jax.lax.dot_general,
                           dimension_numbers=(((1,), (0,)), ((), ())),
                           precision=lax.Precision.DEFAULT,
                           preferred_element_type=jnp.float32)
    g = mm(g8, wk2f_ref[...])
    ari = ari_ref[...]
    arj = g * bjg_ref[...]
    m = ari + arj
    res = ((w1a_ref, b1a_ref, w2a_ref, b2a_ref),
           (w1b_ref, b1b_ref, w2b_ref, b2b_ref))
    for w1, b1, w2, b2 in res:
        t = mm(_ssp(m), w1[...]) + b1[...]
        m = m + mm(t, w2[...]) + b2[...]
    m = _ssp(m)
    new = u_ref[...] * ari + mm(m, wd_ref[...]) + bd_ref[...]
    for w1, b1, w2, b2 in res:
        t = mm(_ssp(new), w1[...]) + b1[...]
        new = new + mm(t, w2[...]) + b2[...]
    out_ref[...] = new


def _edge_chain(ari, bjg, d2, wk2f8, u, wd, bd, w1a, b1a, w2a, b2a,
                w1b, b1b, w2b, b2b):
    full = lambda shape: pl.BlockSpec(shape, lambda i: (0,) * len(shape))
    return pl.pallas_call(
        _edge_body,
        grid=(N_TILES,),
        in_specs=[
            pl.BlockSpec((TILE, F), lambda i: (i, 0)),
            pl.BlockSpec((TILE, F), lambda i: (i, 0)),
            pl.BlockSpec((8, F), lambda i: (i, 0)),
            full((8, F)), full((1, F)), full((F, F)), full((1, F)),
            full((F, F)), full((1, F)), full((F, F)), full((1, F)),
            full((F, F)), full((1, F)), full((F, F)), full((1, F)),
        ],
        out_specs=pl.BlockSpec((TILE, F), lambda i: (i, 0)),
        out_shape=jax.ShapeDtypeStruct((EP, F), jnp.float32),
    )(ari, bjg, d2.reshape(EP // F, F), wk2f8, u, wd, bd,
      w1a, b1a, w2a, b2a, w1b, b1b, w2b, b2b)


def kernel(Z, pos, edge_index, emb_table, params):
    src = edge_index[0].astype(jnp.int32)
    dst = edge_index[1].astype(jnp.int32)
    pad = jnp.full((EP - N_EDGES,), N_NODES, jnp.int32)
    srcp = jnp.concatenate([src, pad])
    dstp = jnp.concatenate([dst, pad])
    posp = jnp.pad(pos.astype(jnp.float32), ((0, NP - N_NODES), (0, 0)))
    px, py, pz = posp[:, 0], posp[:, 1], posp[:, 2]
    z2d = Z.astype(jnp.int32).reshape(N_NODES, 1)
    emb_pad = jnp.pad(emb_table, ((0, 24 - emb_table.shape[0]), (0, 0)))
    zero_rows = jnp.zeros((CHUNK, F), jnp.float32)

    def block_edges(a, b, p):
        wk2f8 = jnp.pad(p["Wk2f"], ((0, 8 - K_RBF), (0, 0)))
        ari, bjg, d2 = _sc_gather(a, b, srcp, dstp, px, py, pz)
        new = _edge_chain(
            ari, bjg, d2, wk2f8, p["u"].reshape(1, F), p["Wd"],
            p["bd"].reshape(1, F),
            p["res"][0]["W1"], p["res"][0]["b1"].reshape(1, F),
            p["res"][0]["W2"], p["res"][0]["b2"].reshape(1, F),
            p["res"][1]["W1"], p["res"][1]["b1"].reshape(1, F),
            p["res"][1]["W2"], p["res"][1]["b2"].reshape(1, F))
        return _sc_scatter(new, dstp, zero_rows)

    p0, p1 = params[0], params[1]
    a, b = _node_embed(z2d, emb_pad, p0["Wi"], p0["bi"].reshape(1, F),
                       p0["Wj"], p0["bj"].reshape(1, F))
    parts = block_edges(a, b, p0)
    a, b = _node_parts(parts, p1["Wi"], p1["bi"].reshape(1, F),
                       p1["Wj"], p1["bj"].reshape(1, F))
    parts = block_edges(a, b, p1)
    return _final_sum(parts)
